# baseline (speedup 1.0000x reference)
import jax
import jax.numpy as jnp
from jax.experimental import pallas as pl
from jax.experimental.pallas import tpu as pltpu

MAX_LEN = 1024
DIM = 256
T = 1024
EPAD = 2048
CH = 4
NCHUNK = 16 // CH
NBUF = 2


def _body(e_ref, x_hbm, o_hbm, rs_ref, inbuf, outbuf, in_sems, out_sems):
    def in_copy(i):
        return pltpu.make_async_copy(
            x_hbm.at[pl.ds(i * CH, CH)], inbuf.at[i % NBUF],
            in_sems.at[i % NBUF])

    def out_copy(i):
        return pltpu.make_async_copy(
            outbuf.at[i % NBUF], o_hbm.at[pl.ds(i * CH, CH)],
            out_sems.at[i % NBUF])

    for i in range(NBUF):
        in_copy(i).start()

    ii = jax.lax.broadcasted_iota(jnp.int32, (T, EPAD), 0)
    kk = jax.lax.broadcasted_iota(jnp.int32, (T, EPAD), 1)
    ss = ii + kk
    w = jnp.where((ss >= MAX_LEN - 1) & (ss <= 2 * MAX_LEN - 2), 1.0, 0.0)
    rs_ref[:] = jnp.dot(w.astype(jnp.float32), e_ref[:],
                        preferred_element_type=jnp.float32)

    for i in range(NCHUNK):
        s = i % NBUF
        in_copy(i).wait()
        if i >= NBUF:
            out_copy(i - NBUF).wait()
        outbuf[s] = inbuf[s] + rs_ref[:][None]
        out_copy(i).start()
        if i + NBUF < NCHUNK:
            in_copy(i + NBUF).start()

    for i in range(NCHUNK - NBUF, NCHUNK):
        out_copy(i).wait()


def kernel(x, rel_embedding):
    b, t, d = x.shape
    e_pad = jnp.concatenate(
        [rel_embedding, jnp.zeros((1, d), rel_embedding.dtype)], axis=0)

    return pl.pallas_call(
        _body,
        in_specs=[
            pl.BlockSpec((EPAD, d), lambda: (0, 0)),
            pl.BlockSpec(memory_space=pl.ANY),
        ],
        out_specs=pl.BlockSpec(memory_space=pl.ANY),
        out_shape=jax.ShapeDtypeStruct((b, t, d), x.dtype),
        scratch_shapes=[
            pltpu.VMEM((t, d), jnp.float32),
            pltpu.VMEM((NBUF, CH, t, d), jnp.float32),
            pltpu.VMEM((NBUF, CH, t, d), jnp.float32),
            pltpu.SemaphoreType.DMA((NBUF,)),
            pltpu.SemaphoreType.DMA((NBUF,)),
        ],
    )(e_pad, x)


# speedup vs baseline: 1.0711x; 1.0711x over previous
import jax
import jax.numpy as jnp
from jax.experimental import pallas as pl
from jax.experimental.pallas import tpu as pltpu

MAX_LEN = 1024
DIM = 256
T = 1024
EPAD = 2048
CH = 1
NCHUNK = 16 // CH
NBUF = 6


def _body(e_ref, x_hbm, o_hbm, rs_ref, inbuf, outbuf, in_sems, out_sems):
    def in_copy(i):
        return pltpu.make_async_copy(
            x_hbm.at[pl.ds(i * CH, CH)], inbuf.at[i % NBUF],
            in_sems.at[i % NBUF])

    def out_copy(i):
        return pltpu.make_async_copy(
            outbuf.at[i % NBUF], o_hbm.at[pl.ds(i * CH, CH)],
            out_sems.at[i % NBUF])

    for i in range(NBUF):
        in_copy(i).start()

    ii = jax.lax.broadcasted_iota(jnp.int32, (T, EPAD), 0)
    kk = jax.lax.broadcasted_iota(jnp.int32, (T, EPAD), 1)
    ss = ii + kk
    w = jnp.where((ss >= MAX_LEN - 1) & (ss <= 2 * MAX_LEN - 2), 1.0, 0.0)
    rs_ref[:] = jnp.dot(w.astype(jnp.float32), e_ref[:],
                        preferred_element_type=jnp.float32)

    for i in range(NCHUNK):
        s = i % NBUF
        in_copy(i).wait()
        if i >= NBUF:
            out_copy(i - NBUF).wait()
        outbuf[s] = inbuf[s] + rs_ref[:][None]
        out_copy(i).start()
        if i + NBUF < NCHUNK:
            in_copy(i + NBUF).start()

    for i in range(NCHUNK - NBUF, NCHUNK):
        out_copy(i).wait()


def kernel(x, rel_embedding):
    b, t, d = x.shape
    e_pad = jnp.concatenate(
        [rel_embedding, jnp.zeros((1, d), rel_embedding.dtype)], axis=0)

    return pl.pallas_call(
        _body,
        in_specs=[
            pl.BlockSpec((EPAD, d), lambda: (0, 0)),
            pl.BlockSpec(memory_space=pl.ANY),
        ],
        out_specs=pl.BlockSpec(memory_space=pl.ANY),
        out_shape=jax.ShapeDtypeStruct((b, t, d), x.dtype),
        scratch_shapes=[
            pltpu.VMEM((t, d), jnp.float32),
            pltpu.VMEM((NBUF, CH, t, d), jnp.float32),
            pltpu.VMEM((NBUF, CH, t, d), jnp.float32),
            pltpu.SemaphoreType.DMA((NBUF,)),
            pltpu.SemaphoreType.DMA((NBUF,)),
        ],
    )(e_pad, x)


# speedup vs baseline: 1.0887x; 1.0164x over previous
import jax
import jax.numpy as jnp
from jax.experimental import pallas as pl
from jax.experimental.pallas import tpu as pltpu

MAX_LEN = 1024
DIM = 256
T = 1024
EPAD = 2048
BB = 8


def _fused_body(e_ref, x_ref, o_ref, rs_ref):
    @pl.when(pl.program_id(0) == 0)
    def _():
        i = jax.lax.broadcasted_iota(jnp.int32, (T, EPAD), 0)
        k = jax.lax.broadcasted_iota(jnp.int32, (T, EPAD), 1)
        s = i + k
        w = jnp.where((s >= MAX_LEN - 1) & (s <= 2 * MAX_LEN - 2), 1.0, 0.0)
        rs_ref[:] = jnp.dot(w.astype(jnp.float32), e_ref[:],
                            preferred_element_type=jnp.float32)

    o_ref[:] = x_ref[:] + rs_ref[:][None]


def kernel(x, rel_embedding):
    b, t, d = x.shape
    e_pad = jnp.concatenate(
        [rel_embedding, jnp.zeros((1, d), rel_embedding.dtype)], axis=0)

    return pl.pallas_call(
        _fused_body,
        grid=(b // BB,),
        in_specs=[
            pl.BlockSpec((EPAD, d), lambda i: (0, 0)),
            pl.BlockSpec((BB, t, d), lambda i: (i, 0, 0)),
        ],
        out_specs=pl.BlockSpec((BB, t, d), lambda i: (i, 0, 0)),
        out_shape=jax.ShapeDtypeStruct((b, t, d), x.dtype),
        scratch_shapes=[pltpu.VMEM((t, d), jnp.float32)],
    )(e_pad, x)
